# baseline (speedup 1.0000x reference)
import functools

import jax
import jax.numpy as jnp
from jax.experimental import pallas as pl
from jax.experimental.pallas import tpu as pltpu

_MAX_ANS_LEN = 30
_EPS = 1e-8
_NEG = -10000.0
_HALVES = 2
_SPLIT = 4
_LANES = 256


def _dcr_kernel(idxs_ref, s0_ref, s1_ref, s2_ref, s3_ref, mv_ref, ei_ref,
                a_s, b_s, n2_s, q_s):
    bi = pl.program_id(0)
    h = pl.program_id(1)
    refs = (s0_ref, s1_ref, s2_ref, s3_ref)
    H = s0_ref.shape[3]
    C = a_s.shape[0]
    sep0 = idxs_ref[bi, 0]
    sep1 = idxs_ref[bi, 1]

    @pl.when(h == 0)
    def _():
        q_s[0:1, :] = s0_ref[0, 0, 1:2, :]
        q_s[1:2, :] = s0_ref[0, 0, pl.ds(sep0 - 1, 1), :]

    q = q_s[...]
    dn = (((1,), (1,)), ((), ()))
    ones = jnp.ones((1, H), jnp.float32)
    for k, r in enumerate(refs):
        chunk = r[0, 0]
        ab = jax.lax.dot_general(q, chunk, dimension_numbers=dn,
                                 preferred_element_type=jnp.float32)
        n2c = jax.lax.dot_general(ones, chunk * chunk, dimension_numbers=dn,
                                  preferred_element_type=jnp.float32)
        row = h * _SPLIT + k
        a_s[pl.ds(row, 1), :] = ab[0:1, :]
        b_s[pl.ds(row, 1), :] = ab[1:2, :]
        n2_s[pl.ds(row, 1), :] = n2c

    @pl.when(h == _HALVES - 1)
    def _():
        qv = q_s[...]
        qn = jnp.sqrt(jnp.sum(qv * qv))
        inv_qn = 1.0 / jnp.maximum(qn, _EPS)

        a2 = a_s[...]
        b2 = b_s[...]
        n2 = n2_s[...]
        pad_row = jnp.ones((1, _LANES), jnp.float32)
        b_dw = jnp.concatenate(
            [b2, jnp.concatenate([b2[1:, :], pad_row], axis=0)], axis=1)
        n2_dw = jnp.concatenate(
            [n2, jnp.concatenate([n2[1:, :], pad_row], axis=0)], axis=1)

        s_iota = jax.lax.broadcasted_iota(jnp.int32, (C, _LANES), 0)
        l_iota = jax.lax.broadcasted_iota(jnp.int32, (C, _LANES), 1)
        i_idx = s_iota * _LANES + l_iota

        mv = jnp.full((C, _LANES), _NEG, jnp.float32)
        best_o = jnp.zeros((C, _LANES), jnp.int32)
        for o in range(_MAX_ANS_LEN):
            b_o = jax.lax.slice(b_dw, (0, o), (C, o + _LANES))
            n2_o = jax.lax.slice(n2_dw, (0, o), (C, o + _LANES))
            num = a2 + b_o
            r = jnp.minimum(jax.lax.rsqrt(n2 + n2_o), 1.0 / _EPS)
            sim = num * r * inv_qn
            valid = i_idx < (sep1 - o)
            sim = jnp.where(valid, sim, _NEG)
            if o == 0:
                mv = sim
            else:
                upd = sim > mv
                mv = jnp.where(upd, sim, mv)
                best_o = jnp.where(upd, o, best_o)

        i_valid = (i_idx > sep0) & (i_idx < sep1)
        mv_ref[0] = jnp.where(i_valid, mv, _NEG)
        ei_ref[0] = jnp.where(i_valid, i_idx + best_o, -1)


@functools.partial(jax.jit, static_argnames=())
def kernel(sequence_outputs, idxs):
    B, S, H = sequence_outputs.shape
    C = S // _LANES
    out_shape = (
        jax.ShapeDtypeStruct((B, C, _LANES), jnp.float32),
        jax.ShapeDtypeStruct((B, C, _LANES), jnp.int32),
    )
    seqv = sequence_outputs.reshape(B, C, _LANES, H)
    specs = [
        pl.BlockSpec((1, 1, _LANES, H), functools.partial(
            lambda k, b, h: (b, h * _SPLIT + k, 0, 0), k))
        for k in range(_SPLIT)
    ]
    mv, ei = pl.pallas_call(
        _dcr_kernel,
        grid=(B, _HALVES),
        in_specs=[pl.BlockSpec(memory_space=pltpu.SMEM)] + specs,
        out_specs=(
            pl.BlockSpec((1, C, _LANES), lambda b, h: (b, 0, 0)),
            pl.BlockSpec((1, C, _LANES), lambda b, h: (b, 0, 0)),
        ),
        out_shape=out_shape,
        scratch_shapes=[
            pltpu.VMEM((C, _LANES), jnp.float32),
            pltpu.VMEM((C, _LANES), jnp.float32),
            pltpu.VMEM((C, _LANES), jnp.float32),
            pltpu.VMEM((2, H), jnp.float32),
        ],
        compiler_params=pltpu.CompilerParams(
            dimension_semantics=("arbitrary", "arbitrary"),
        ),
    )(idxs, *([seqv] * _SPLIT))
    return mv.reshape(B, S), ei.reshape(B, S)
